# baseline (speedup 1.0000x reference)
import numpy as np
import jax
import jax.numpy as jnp
from jax.experimental import pallas as pl
from jax.experimental.pallas import tpu as pltpu

_NUM_ANGLE = 100
_NUM_RHO = 100
_H = 100
_W = 100
_P = _H * _W
_A_BLK = 8
_A_STEPS = 13
_A_SUB = 4
_K_SUB = _A_SUB * _NUM_RHO
_N_SUB = _A_STEPS * _A_BLK // _A_SUB


def _rk_table() -> np.ndarray:
    irho = float(int(np.sqrt(_H * _H + _W * _W) + 1)) / float(_NUM_RHO - 1)
    itheta = np.pi / _NUM_ANGLE
    angles = np.arange(_NUM_ANGLE, dtype=np.float64) * itheta
    tab_cos = (np.cos(angles) / irho).astype(np.float32)
    tab_sin = (np.sin(angles) / irho).astype(np.float32)
    ys, xs = np.meshgrid(np.arange(_H), np.arange(_W), indexing="ij")
    xx = (xs - (_W // 2)).reshape(-1).astype(np.float32)
    yy = (ys - (_H // 2)).reshape(-1).astype(np.float32)
    proj = xx[None, :] * tab_cos[:, None] + yy[None, :] * tab_sin[:, None]
    r = np.where(proj >= 0,
                 np.floor(proj + np.float32(0.5)),
                 np.ceil(proj - np.float32(0.5))).astype(np.int32) + _NUM_RHO // 2
    r = np.clip(r, 0, _NUM_RHO - 1)
    rk = r + (np.arange(_NUM_ANGLE, dtype=np.int32) * _NUM_RHO)[:, None]
    rk_pad = np.full((_N_SUB * _A_SUB, _P), -1, dtype=np.int32)
    rk_pad[:_NUM_ANGLE] = rk
    out = np.full((_N_SUB, 8, _P), -1, dtype=np.int32)
    out[:, :_A_SUB, :] = rk_pad.reshape(_N_SUB, _A_SUB, _P)
    return out


_RK = _rk_table()


def _hough_body(rk_ref, f_ref, o_ref, oh_scr):
    g = pl.program_id(0)

    @pl.when(g < _N_SUB)
    def _gen():
        rk = rk_ref[0, :_A_SUB, :]
        rk_e = jnp.broadcast_to(
            rk[:, None, :], (_A_SUB, _NUM_RHO, _P)
        ).reshape(_K_SUB, _P)
        kcol = g * _K_SUB + jax.lax.broadcasted_iota(
            jnp.int32, (_K_SUB, _P), 0)
        oh_scr[jax.lax.rem(g, 2)] = (rk_e == kcol).astype(jnp.bfloat16)

    @pl.when(g > 0)
    def _dot():
        s = g - 1
        oh = oh_scr[jax.lax.rem(s, 2)]
        acc = jax.lax.dot_general(
            f_ref[...], oh, (((1,), (1,)), ((), ())),
            preferred_element_type=jnp.float32)
        acc = acc.reshape(acc.shape[0], _A_SUB, _NUM_RHO)
        m = jax.lax.rem(s, 2)

        @pl.when(m == 0)
        def _():
            o_ref[:, :_A_SUB, :] = acc

        @pl.when(m == 1)
        def _():
            o_ref[:, _A_SUB:, :] = acc


def kernel(feat):
    n, c, h, w = feat.shape
    nc = n * c
    feat2d = feat.reshape(nc, _P).astype(jnp.bfloat16)
    rk = jnp.asarray(_RK)

    out = pl.pallas_call(
        _hough_body,
        grid=(_N_SUB + 1,),
        in_specs=[
            pl.BlockSpec((1, 8, _P), lambda g: (jnp.minimum(g, _N_SUB - 1), 0, 0)),
            pl.BlockSpec((nc, _P), lambda g: (0, 0)),
        ],
        out_specs=pl.BlockSpec(
            (nc, _A_BLK, _NUM_RHO),
            lambda g: (0, jnp.maximum(g - 1, 0) // 2, 0)),
        out_shape=jax.ShapeDtypeStruct((nc, _NUM_ANGLE, _NUM_RHO), jnp.float32),
        scratch_shapes=[pltpu.VMEM((2, _K_SUB, _P), jnp.bfloat16)],
        compiler_params=pltpu.CompilerParams(
            dimension_semantics=("arbitrary",),
        ),
    )(rk, feat2d)

    return out.reshape(n, c, _NUM_ANGLE, _NUM_RHO)


# speedup vs baseline: 1.1689x; 1.1689x over previous
import numpy as np
import jax
import jax.numpy as jnp
from jax.experimental import pallas as pl
from jax.experimental.pallas import tpu as pltpu

_NUM_ANGLE = 100
_NUM_RHO = 100
_H = 100
_W = 100
_P = _H * _W
_A_BLK = 8
_A_STEPS = 13
_N_HALF = 2
_A_SUB = _A_BLK // _N_HALF
_K_SUB = _A_SUB * _NUM_RHO


def _rk_table() -> np.ndarray:
    irho = float(int(np.sqrt(_H * _H + _W * _W) + 1)) / float(_NUM_RHO - 1)
    itheta = np.pi / _NUM_ANGLE
    angles = np.arange(_NUM_ANGLE, dtype=np.float64) * itheta
    tab_cos = (np.cos(angles) / irho).astype(np.float32)
    tab_sin = (np.sin(angles) / irho).astype(np.float32)
    ys, xs = np.meshgrid(np.arange(_H), np.arange(_W), indexing="ij")
    xx = (xs - (_W // 2)).reshape(-1).astype(np.float32)
    yy = (ys - (_H // 2)).reshape(-1).astype(np.float32)
    proj = xx[None, :] * tab_cos[:, None] + yy[None, :] * tab_sin[:, None]
    r = np.where(proj >= 0,
                 np.floor(proj + np.float32(0.5)),
                 np.ceil(proj - np.float32(0.5))).astype(np.int32) + _NUM_RHO // 2
    r = np.clip(r, 0, _NUM_RHO - 1)
    rk = r + (np.arange(_NUM_ANGLE, dtype=np.int32) * _NUM_RHO)[:, None]
    out = np.full((_A_STEPS * _A_BLK, _P), -1, dtype=np.int32)
    out[:_NUM_ANGLE] = rk
    return out


_RK = _rk_table()


def _hough_body(rk_ref, f_ref, o_ref):
    i = pl.program_id(0)
    rk = rk_ref[...]
    f = f_ref[...]

    def gen(half):
        sub = rk[half * _A_SUB:(half + 1) * _A_SUB, :]
        rk_e = jnp.broadcast_to(
            sub[:, None, :], (_A_SUB, _NUM_RHO, _P)
        ).reshape(_K_SUB, _P)
        kcol = (i * (_A_BLK * _NUM_RHO) + half * _K_SUB
                + jax.lax.broadcasted_iota(jnp.int32, (_K_SUB, _P), 0))
        return (rk_e == kcol).astype(jnp.bfloat16)

    def dot(oh):
        acc = jax.lax.dot_general(
            f, oh, (((1,), (1,)), ((), ())),
            preferred_element_type=jnp.float32)
        return acc.reshape(acc.shape[0], _A_SUB, _NUM_RHO)

    oh0 = gen(0)
    acc0 = dot(oh0)
    oh1 = gen(1)
    acc1 = dot(oh1)
    o_ref[:, :_A_SUB, :] = acc0
    o_ref[:, _A_SUB:, :] = acc1


def kernel(feat):
    n, c, h, w = feat.shape
    nc = n * c
    feat2d = feat.reshape(nc, _P).astype(jnp.bfloat16)
    rk = jnp.asarray(_RK)

    out = pl.pallas_call(
        _hough_body,
        grid=(_A_STEPS,),
        in_specs=[
            pl.BlockSpec((_A_BLK, _P), lambda i: (i, 0)),
            pl.BlockSpec((nc, _P), lambda i: (0, 0)),
        ],
        out_specs=pl.BlockSpec((nc, _A_BLK, _NUM_RHO), lambda i: (0, i, 0)),
        out_shape=jax.ShapeDtypeStruct((nc, _NUM_ANGLE, _NUM_RHO), jnp.float32),
        compiler_params=pltpu.CompilerParams(
            dimension_semantics=("arbitrary",),
        ),
    )(rk, feat2d)

    return out.reshape(n, c, _NUM_ANGLE, _NUM_RHO)
